# baseline (speedup 1.0000x reference)
import functools

import jax
import jax.numpy as jnp
from jax import lax
from jax.experimental import pallas as pl
from jax.experimental.pallas import tpu as pltpu
from jax.experimental.pallas import tpu_sc as plsc

L = 4096
NB = 65
HIST_PAD = 4240
NLANE = 16
NVREG = L // NLANE
NCORES = 2
NSUB = 16
NWORKERS = NCORES * NSUB
RADIX = 1 << 11

_I32_MIN = -2147483648
_I32_MAX = 2147483647


def _row_kernel(x_hbm, out_hbm,
                xvA, keys0A, pay0A, keys1A, pay1A, hist0A, hist1A, binsA,
                rowhistA,
                xvB, keys0B, pay0B, keys1B, pay1B, hist0B, hist1B, binsB,
                rowhistB,
                semA, semB, osemA, osemB):
  total_rows = x_hbm.shape[0]
  rows_per_worker = total_rows // NWORKERS
  npairs = rows_per_worker // 2
  wid = lax.axis_index("s") * NCORES + lax.axis_index("c")
  iota = lax.iota(jnp.int32, NLANE)
  zeros16i = jnp.zeros((NLANE,), jnp.int32)
  zeros16f = jnp.zeros((NLANE,), jnp.float32)

  binsA[pl.ds(L, NLANE)] = zeros16i
  binsB[pl.ds(L, NLANE)] = zeros16i

  def pair_body(r, _):
    rowA = wid * rows_per_worker + 2 * r
    rowB = rowA + 1

    @pl.when(r > 0)
    def _():
      pltpu.make_async_copy(rowhistA, out_hbm.at[rowA - 2], osemA).wait()
      pltpu.make_async_copy(rowhistB, out_hbm.at[rowB - 2], osemB).wait()

    cpA = pltpu.make_async_copy(x_hbm.at[rowA], xvA, semA)
    cpB = pltpu.make_async_copy(x_hbm.at[rowB], xvB, semB)
    cpA.start()
    cpB.start()
    cpA.wait()
    cpB.wait()

    def h0clear_body(j, _):
      hist0A[pl.ds(j * NLANE, NLANE)] = zeros16i
      hist0B[pl.ds(j * NLANE, NLANE)] = zeros16i
      return 0

    lax.fori_loop(0, RADIX // NLANE, h0clear_body, 0, unroll=4)

    def keyfast_body(j, carry):
      zA, zB = carry
      idxv = j * NLANE + iota
      vA = xvA[pl.ds(j * NLANE, NLANE)]
      vB = xvB[pl.ds(j * NLANE, NLANE)]
      tA = plsc.bitcast(vA, jnp.int32)
      tB = plsc.bitcast(vB, jnp.int32)
      uA = (tA ^ (lax.shift_right_arithmetic(tA, 31) & _I32_MAX)) ^ _I32_MIN
      uB = (tB ^ (lax.shift_right_arithmetic(tB, 31) & _I32_MAX)) ^ _I32_MIN
      dA = uA & (RADIX - 1)
      dB = uB & (RADIX - 1)
      occA, lastA = plsc.scan_count(dA)
      occB, lastB = plsc.scan_count(dB)
      keys0A[pl.ds(j * NLANE, NLANE)] = uA
      keys0B[pl.ds(j * NLANE, NLANE)] = uB
      pay0A[pl.ds(j * NLANE, NLANE)] = idxv
      pay0B[pl.ds(j * NLANE, NLANE)] = idxv
      plsc.addupdate_scatter(hist0A, [dA], occA, mask=lastA)
      plsc.addupdate_scatter(hist0B, [dB], occB, mask=lastB)
      return zA | (tA + tA == 0), zB | (tB + tB == 0)

    zA, zB = lax.fori_loop(0, NVREG, keyfast_body, (iota < 0, iota < 0),
                           unroll=4)

    def make_slow_path(xv, keys0, hist0):
      def slow_path():
        def valid_body(j, carry):
          fv, lv = carry
          v = xv[pl.ds(j * NLANE, NLANE)]
          nz = v != 0.0
          idxv = j * NLANE + iota
          fv = jnp.minimum(fv, jnp.where(nz, idxv, jnp.int32(L)))
          lv = jnp.maximum(lv, jnp.where(nz, idxv, jnp.int32(-1)))
          return fv, lv

        fv, lv = lax.fori_loop(0, NVREG, valid_body,
                               (zeros16i + L, zeros16i - 1), unroll=4)
        s_, e_ = jnp.min(fv), jnp.max(lv)

        def hclear(j, _):
          hist0[pl.ds(j * NLANE, NLANE)] = zeros16i
          return 0

        lax.fori_loop(0, RADIX // NLANE, hclear, 0, unroll=4)

        def keymask_body(j, _):
          u = keys0[pl.ds(j * NLANE, NLANE)]
          idxv = j * NLANE + iota
          ok = (idxv >= s_) & (idxv <= e_)
          key = jnp.where(ok, u, jnp.int32(-1))
          keys0[pl.ds(j * NLANE, NLANE)] = key
          d = key & (RADIX - 1)
          occ, last = plsc.scan_count(d)
          plsc.addupdate_scatter(hist0, [d], occ, mask=last)
          return 0

        lax.fori_loop(0, NVREG, keymask_body, 0, unroll=4)
        return s_, e_

      return slow_path

    full = lambda: (jnp.int32(0), jnp.int32(L - 1))
    anyzeroA = jnp.max(zA.astype(jnp.int32)) > 0
    anyzeroB = jnp.max(zB.astype(jnp.int32)) > 0
    startA, endA = lax.cond(anyzeroA, make_slow_path(xvA, keys0A, hist0A),
                            full)
    startB, endB = lax.cond(anyzeroB, make_slow_path(xvB, keys0B, hist0B),
                            full)
    vlenA = endA - startA + 1
    vlenB = endB - startB + 1
    lencA = jnp.maximum(vlenA, 1)
    lencB = jnp.maximum(vlenB, 1)

    def exclusive_scan2(srcA, srcB, clrA, clrB, n, m):
      fifteen = zeros16i + (NLANE - 1)

      def body(j, carry):
        cA, cB = carry
        vA = srcA[pl.ds(j * NLANE, NLANE)]
        vB = srcB[pl.ds(j * NLANE, NLANE)]
        incA = plsc.cumsum(vA)
        incB = plsc.cumsum(vB)
        srcA[pl.ds(j * NLANE, NLANE)] = incA - vA + cA
        srcB[pl.ds(j * NLANE, NLANE)] = incB - vB + cB

        @pl.when(j < m // NLANE)
        def _():
          clrA[pl.ds(j * NLANE, NLANE)] = zeros16i
          clrB[pl.ds(j * NLANE, NLANE)] = zeros16i

        totA = jnp.take_along_axis(incA, fifteen, axis=0)
        totB = jnp.take_along_axis(incB, fifteen, axis=0)
        return cA + totA, cB + totB

      lax.fori_loop(0, n // NLANE, body, (zeros16i, zeros16i), unroll=2)

    exclusive_scan2(hist0A, hist0B, hist1A, hist1B, RADIX, RADIX)

    def permute01_pair(j, sh, nbits2, Ar, Br):
      kinA, pinA, koutA, poutA, histA, histnextA = Ar
      kinB, pinB, koutB, poutB, histB, histnextB = Br
      sl = pl.ds(j * NLANE, NLANE)
      kA = kinA[sl]
      kB = kinB[sl]
      pA = pinA[sl]
      pB = pinB[sl]
      dA = lax.shift_right_logical(kA, sh) & (RADIX - 1)
      dB = lax.shift_right_logical(kB, sh) & (RADIX - 1)
      dnA = lax.shift_right_logical(kA, sh + 11) & ((1 << nbits2) - 1)
      dnB = lax.shift_right_logical(kB, sh + 11) & ((1 << nbits2) - 1)
      occA, lastA = plsc.scan_count(dA)
      occB, lastB = plsc.scan_count(dB)
      occnA, lastnA = plsc.scan_count(dnA)
      occnB, lastnB = plsc.scan_count(dnB)
      baseA = plsc.load_gather(histA, [dA])
      baseB = plsc.load_gather(histB, [dB])
      slotA = baseA + occA - 1
      slotB = baseB + occB - 1
      plsc.store_scatter(koutA, [slotA], kA)
      plsc.store_scatter(koutB, [slotB], kB)
      plsc.store_scatter(poutA, [slotA], pA)
      plsc.store_scatter(poutB, [slotB], pB)
      plsc.addupdate_scatter(histA, [dA], occA, mask=lastA)
      plsc.addupdate_scatter(histB, [dB], occB, mask=lastB)
      plsc.addupdate_scatter(histnextA, [dnA], occnA, mask=lastnA)
      plsc.addupdate_scatter(histnextB, [dnB], occnB, mask=lastnB)
      return 0

    def permute0_body(j, _):
      return permute01_pair(
          j, 0, 11,
          (keys0A, pay0A, keys1A, pay1A, hist0A, hist1A),
          (keys0B, pay0B, keys1B, pay1B, hist0B, hist1B))

    lax.fori_loop(0, NVREG, permute0_body, 0, unroll=4)

    exclusive_scan2(hist1A, hist1B, hist0A, hist0B, RADIX, 1024)

    def permute1_body(j, _):
      return permute01_pair(
          j, 11, 10,
          (keys1A, pay1A, keys0A, pay0A, hist1A, hist0A),
          (keys1B, pay1B, keys0B, pay0B, hist1B, hist0B))

    lax.fori_loop(0, NVREG, permute1_body, 0, unroll=4)

    exclusive_scan2(hist0A, hist0B, hist1A, hist1B, 1024, 0)

    def hclear_body(j, _):
      rowhistA[pl.ds(j * NLANE, NLANE)] = zeros16f
      rowhistB[pl.ds(j * NLANE, NLANE)] = zeros16f
      return 0

    lax.fori_loop(0, HIST_PAD // NLANE, hclear_body, 0, unroll=4)

    invlenA = (zeros16f + 1.0) / (zeros16i + lencA).astype(jnp.float32)
    invlenB = (zeros16f + 1.0) / (zeros16i + lencB).astype(jnp.float32)

    def permute2_body(j, _):
      sl = pl.ds(j * NLANE, NLANE)
      kA = keys0A[sl]
      kB = keys0B[sl]
      pA = pay0A[sl]
      pB = pay0B[sl]
      dA = lax.shift_right_logical(kA, 22) & 1023
      dB = lax.shift_right_logical(kB, 22) & 1023
      occA, lastA = plsc.scan_count(dA)
      occB, lastB = plsc.scan_count(dB)
      baseA = plsc.load_gather(hist0A, [dA])
      baseB = plsc.load_gather(hist0B, [dB])
      slotA = baseA + occA - 1
      slotB = baseB + occB - 1
      plsc.addupdate_scatter(hist0A, [dA], occA, mask=lastA)
      plsc.addupdate_scatter(hist0B, [dB], occB, mask=lastB)
      bfA = (slotA * NB).astype(jnp.float32) * invlenA + 5e-5
      bfB = (slotB * NB).astype(jnp.float32) * invlenB + 5e-5
      bA = jnp.minimum(bfA.astype(jnp.int32), NB - 1)
      bB = jnp.minimum(bfB.astype(jnp.int32), NB - 1)
      plsc.store_scatter(binsA, [pA], bA)
      plsc.store_scatter(binsB, [pB], bB)
      return 0

    lax.fori_loop(0, NVREG, permute2_body, 0, unroll=4)

    invA = (zeros16f + 1.0) / \
        (zeros16i + jnp.maximum(vlenA - 1, 1)).astype(jnp.float32)
    invB = (zeros16f + 1.0) / \
        (zeros16i + jnp.maximum(vlenB - 1, 1)).astype(jnp.float32)

    def trans_masked_one(j, bins, rowhist, start, end, inv):
      a = bins[pl.ds(j * NLANE, NLANE)]
      b = bins[pl.ds(j * NLANE + 1, NLANE)]
      t = j * NLANE + iota
      ok = (t >= start) & (t <= end - 1)
      cell = a * NB + b
      occ, last = plsc.scan_count(cell, mask=ok)
      plsc.addupdate_scatter(rowhist, [cell], occ.astype(jnp.float32) * inv,
                             mask=last & ok)
      return 0

    def trans_all_masked():
      def body(j, _):
        trans_masked_one(j, binsA, rowhistA, startA, endA, invA)
        trans_masked_one(j, binsB, rowhistB, startB, endB, invB)
        return 0

      lax.fori_loop(0, NVREG, body, 0, unroll=2)
      return 0

    def trans_all_fast():
      def body(j, _):
        sl = pl.ds(j * NLANE, NLANE)
        sl1 = pl.ds(j * NLANE + 1, NLANE)
        aA = binsA[sl]
        aB = binsB[sl]
        bA = binsA[sl1]
        bB = binsB[sl1]
        cellA = aA * NB + bA
        cellB = aB * NB + bB
        occA, lastA = plsc.scan_count(cellA)
        occB, lastB = plsc.scan_count(cellB)
        plsc.addupdate_scatter(rowhistA, [cellA],
                               occA.astype(jnp.float32) * invA, mask=lastA)
        plsc.addupdate_scatter(rowhistB, [cellB],
                               occB.astype(jnp.float32) * invB, mask=lastB)
        return 0

      lax.fori_loop(0, NVREG - 1, body, 0, unroll=4)
      trans_masked_one(NVREG - 1, binsA, rowhistA, startA, endA, invA)
      trans_masked_one(NVREG - 1, binsB, rowhistB, startB, endB, invB)
      return 0

    lax.cond(anyzeroA | anyzeroB, trans_all_masked, trans_all_fast)

    pltpu.make_async_copy(rowhistA, out_hbm.at[rowA], osemA).start()
    pltpu.make_async_copy(rowhistB, out_hbm.at[rowB], osemB).start()
    return 0

  lax.fori_loop(0, npairs, pair_body, 0)
  last_rowA = wid * rows_per_worker + 2 * (npairs - 1)
  pltpu.make_async_copy(rowhistA, out_hbm.at[last_rowA], osemA).wait()
  pltpu.make_async_copy(rowhistB, out_hbm.at[last_rowA + 1], osemB).wait()


@jax.jit
def kernel(x):
  N, C, Lx = x.shape
  rows = N * C
  x2 = x.reshape(rows, Lx)
  mesh = plsc.VectorSubcoreMesh(core_axis_name="c", subcore_axis_name="s",
                                num_cores=NCORES, num_subcores=NSUB)
  per_row_scratch = [
      pltpu.VMEM((L,), jnp.float32),
      pltpu.VMEM((L,), jnp.int32),
      pltpu.VMEM((L,), jnp.int32),
      pltpu.VMEM((L,), jnp.int32),
      pltpu.VMEM((L,), jnp.int32),
      pltpu.VMEM((RADIX,), jnp.int32),
      pltpu.VMEM((RADIX,), jnp.int32),
      pltpu.VMEM((L + NLANE,), jnp.int32),
      pltpu.VMEM((HIST_PAD,), jnp.float32),
  ]
  run = functools.partial(
      pl.kernel,
      mesh=mesh,
      compiler_params=pltpu.CompilerParams(needs_layout_passes=False),
      out_type=jax.ShapeDtypeStruct((rows, HIST_PAD), jnp.float32),
      scratch_types=per_row_scratch + per_row_scratch + [
          pltpu.SemaphoreType.DMA,
          pltpu.SemaphoreType.DMA,
          pltpu.SemaphoreType.DMA,
          pltpu.SemaphoreType.DMA,
      ],
  )(_row_kernel)
  out = run(x2)
  return out[:, :NB * NB].reshape(N, C, NB, NB)


# speedup vs baseline: 1.5594x; 1.5594x over previous
import functools

import jax
import jax.numpy as jnp
from jax import lax
from jax.experimental import pallas as pl
from jax.experimental.pallas import tpu as pltpu
from jax.experimental.pallas import tpu_sc as plsc

L = 4096
NB = 65
HIST_PAD = 4240
NLANE = 16
NVREG = L // NLANE
NCORES = 2
NSUB = 16
NWORKERS = NCORES * NSUB
RAD0 = 1 << 11
RAD1 = 1 << 10
RAD2 = 1 << 11
NROWS = 4

_I32_MIN = -2147483648
_I32_MAX = 2147483647
_NREFS = 8


def _row_kernel(x_hbm, out_hbm, *scratch):
  total_rows = x_hbm.shape[0]
  rows_per_worker = total_rows // NWORKERS
  ngroups = rows_per_worker // NROWS
  wid = lax.axis_index("s") * NCORES + lax.axis_index("c")
  iota = lax.iota(jnp.int32, NLANE)
  zeros16i = jnp.zeros((NLANE,), jnp.int32)
  zeros16f = jnp.zeros((NLANE,), jnp.float32)
  R = range(NROWS)

  xv = [scratch[i * _NREFS + 0] for i in R]
  keys0 = [scratch[i * _NREFS + 1] for i in R]
  pay0 = [scratch[i * _NREFS + 2] for i in R]
  keys1 = [scratch[i * _NREFS + 3] for i in R]
  pay1 = [scratch[i * _NREFS + 4] for i in R]
  hist0 = [scratch[i * _NREFS + 5] for i in R]
  hist1 = [scratch[i * _NREFS + 6] for i in R]
  bins = [scratch[i * _NREFS + 7] for i in R]
  rowhist = xv
  isem = [scratch[NROWS * _NREFS + i] for i in R]
  osem = [scratch[NROWS * _NREFS + NROWS + i] for i in R]

  for i in R:
    bins[i][pl.ds(L, NLANE)] = zeros16i

  def ds(j):
    return pl.ds(j * NLANE, NLANE)

  def pair_body(r, _):
    base = wid * rows_per_worker + NROWS * r

    @pl.when(r > 0)
    def _():
      for i in R:
        pltpu.make_async_copy(rowhist[i], out_hbm.at[base - NROWS + i],
                              osem[i]).wait()

    cps = [pltpu.make_async_copy(x_hbm.at[base + i],
                                 xv[i].at[pl.ds(0, L)], isem[i]) for i in R]
    for cp in cps:
      cp.start()
    for cp in cps:
      cp.wait()

    def h0clear_body(j, _):
      for i in R:
        hist0[i][ds(j)] = zeros16i
      return 0

    lax.fori_loop(0, RAD0 // NLANE, h0clear_body, 0, unroll=4)

    def keyfast_body(j, zs):
      idxv = j * NLANE + iota
      vs = [xv[i][ds(j)] for i in R]
      ts = [plsc.bitcast(v, jnp.int32) for v in vs]
      us = [(t ^ (lax.shift_right_arithmetic(t, 31) & _I32_MAX)) ^ _I32_MIN
            for t in ts]
      dd = [u & (RAD0 - 1) for u in us]
      sc = [plsc.scan_count(d) for d in dd]
      for i in R:
        keys0[i][ds(j)] = us[i]
      for i in R:
        pay0[i][ds(j)] = idxv
      for i in R:
        plsc.addupdate_scatter(hist0[i], [dd[i]], sc[i][0], mask=sc[i][1])
      return tuple(zs[i] | (ts[i] + ts[i] == 0) for i in R)

    zs = lax.fori_loop(0, NVREG, keyfast_body, tuple(iota < 0 for _ in R),
                       unroll=2)

    def make_slow_path(i):
      def slow_path():
        def valid_body(j, carry):
          fv, lv = carry
          v = xv[i][ds(j)]
          nz = v != 0.0
          idxv = j * NLANE + iota
          fv = jnp.minimum(fv, jnp.where(nz, idxv, jnp.int32(L)))
          lv = jnp.maximum(lv, jnp.where(nz, idxv, jnp.int32(-1)))
          return fv, lv

        fv, lv = lax.fori_loop(0, NVREG, valid_body,
                               (zeros16i + L, zeros16i - 1), unroll=4)
        s_, e_ = jnp.min(fv), jnp.max(lv)

        def hclear(j, _):
          hist0[i][ds(j)] = zeros16i
          return 0

        lax.fori_loop(0, RAD0 // NLANE, hclear, 0, unroll=4)

        def keymask_body(j, _):
          u = keys0[i][ds(j)]
          idxv = j * NLANE + iota
          ok = (idxv >= s_) & (idxv <= e_)
          key = jnp.where(ok, u, jnp.int32(-1))
          keys0[i][ds(j)] = key
          d = key & (RAD0 - 1)
          occ, last = plsc.scan_count(d)
          plsc.addupdate_scatter(hist0[i], [d], occ, mask=last)
          return 0

        lax.fori_loop(0, NVREG, keymask_body, 0, unroll=4)
        return s_, e_

      return slow_path

    full = lambda: (jnp.int32(0), jnp.int32(L - 1))
    anyzero = [jnp.max(z.astype(jnp.int32)) > 0 for z in zs]
    se = [lax.cond(anyzero[i], make_slow_path(i), full) for i in R]
    start = [s for s, _ in se]
    end = [e for _, e in se]
    vlen = [end[i] - start[i] + 1 for i in R]
    lenc = [jnp.maximum(v, 1) for v in vlen]

    def exclusive_scan(srcs, clrs, n, clr_vregs):
      fifteen = zeros16i + (NLANE - 1)
      iters = n // NLANE

      def body(j, carry):
        vs = [src[ds(j)] for src in srcs]
        incs = [plsc.cumsum(v) for v in vs]
        for i in R:
          srcs[i][ds(j)] = incs[i] - vs[i] + carry[i]
        if clr_vregs >= iters:
          k = clr_vregs // iters
          for c in range(k):
            for clr in clrs:
              clr[ds(k * j + c)] = zeros16i
        elif clr_vregs:
          @pl.when(j < clr_vregs)
          def _():
            for clr in clrs:
              clr[ds(j)] = zeros16i
        tots = [jnp.take_along_axis(inc, fifteen, axis=0) for inc in incs]
        return tuple(carry[i] + tots[i] for i in R)

      lax.fori_loop(0, iters, body, tuple(zeros16i for _ in R), unroll=2)

    exclusive_scan(hist0, hist1, RAD0, RAD1 // NLANE)

    def permute01_body(j, sh, nbits, shn, nbitsn, kin, pin, kout, pout,
                       hist, histnext):
      ks = [kin[i][ds(j)] for i in R]
      ps = [pin[i][ds(j)] for i in R]
      dd = [lax.shift_right_logical(k, sh) & ((1 << nbits) - 1) for k in ks]
      dn = [lax.shift_right_logical(k, shn) & ((1 << nbitsn) - 1)
            for k in ks]
      sc = [plsc.scan_count(d) for d in dd]
      scn = [plsc.scan_count(d) for d in dn]
      bases = [plsc.load_gather(hist[i], [dd[i]]) for i in R]
      slots = [bases[i] + sc[i][0] - 1 for i in R]
      for i in R:
        plsc.store_scatter(kout[i], [slots[i]], ks[i])
      for i in R:
        plsc.store_scatter(pout[i], [slots[i]], ps[i])
      for i in R:
        plsc.addupdate_scatter(hist[i], [dd[i]], sc[i][0], mask=sc[i][1])
      for i in R:
        plsc.addupdate_scatter(histnext[i], [dn[i]], scn[i][0],
                               mask=scn[i][1])
      return 0

    lax.fori_loop(
        0, NVREG,
        lambda j, _: permute01_body(j, 0, 11, 11, 10, keys0, pay0, keys1,
                                    pay1, hist0, hist1),
        0, unroll=2)

    exclusive_scan(hist1, hist0, RAD1, RAD0 // NLANE)

    lax.fori_loop(
        0, NVREG,
        lambda j, _: permute01_body(j, 11, 10, 21, 11, keys1, pay1, keys0,
                                    pay0, hist1, hist0),
        0, unroll=2)

    exclusive_scan(hist0, (), RAD2, 0)

    def hclear_body(j, _):
      for i in R:
        rowhist[i][ds(j)] = zeros16f
      return 0

    lax.fori_loop(0, HIST_PAD // NLANE, hclear_body, 0, unroll=4)

    invlen = [(zeros16f + 1.0) / (zeros16i + lenc[i]).astype(jnp.float32)
              for i in R]

    def permute2_body(j, _):
      ks = [keys0[i][ds(j)] for i in R]
      ps = [pay0[i][ds(j)] for i in R]
      dd = [lax.shift_right_logical(k, 21) & (RAD2 - 1) for k in ks]
      sc = [plsc.scan_count(d) for d in dd]
      bases = [plsc.load_gather(hist0[i], [dd[i]]) for i in R]
      slots = [bases[i] + sc[i][0] - 1 for i in R]
      for i in R:
        plsc.addupdate_scatter(hist0[i], [dd[i]], sc[i][0], mask=sc[i][1])
      bf = [(slots[i] * NB).astype(jnp.float32) * invlen[i] + 5e-5
            for i in R]
      bb = [jnp.minimum(b.astype(jnp.int32), NB - 1) for b in bf]
      for i in R:
        plsc.store_scatter(bins[i], [ps[i]], bb[i])
      return 0

    lax.fori_loop(0, NVREG, permute2_body, 0, unroll=2)

    inv = [(zeros16f + 1.0) /
           (zeros16i + jnp.maximum(vlen[i] - 1, 1)).astype(jnp.float32)
           for i in R]

    def trans_masked_one(j, i):
      a = bins[i][ds(j)]
      b = bins[i][pl.ds(j * NLANE + 1, NLANE)]
      t = j * NLANE + iota
      ok = (t >= start[i]) & (t <= end[i] - 1)
      cell = a * NB + b
      occ, last = plsc.scan_count(cell, mask=ok)
      plsc.addupdate_scatter(rowhist[i], [cell],
                             occ.astype(jnp.float32) * inv[i],
                             mask=last & ok)
      return 0

    def trans_all_masked():
      def body(j, _):
        for i in R:
          trans_masked_one(j, i)
        return 0

      lax.fori_loop(0, NVREG, body, 0, unroll=2)
      return 0

    def trans_all_fast():
      def body(j, _):
        aa = [bins[i][ds(j)] for i in R]
        ab = [bins[i][pl.ds(j * NLANE + 1, NLANE)] for i in R]
        cells = [aa[i] * NB + ab[i] for i in R]
        sc = [plsc.scan_count(c) for c in cells]
        for i in R:
          plsc.addupdate_scatter(rowhist[i], [cells[i]],
                                 sc[i][0].astype(jnp.float32) * inv[i],
                                 mask=sc[i][1])
        return 0

      lax.fori_loop(0, NVREG - 1, body, 0, unroll=2)
      for i in R:
        trans_masked_one(NVREG - 1, i)
      return 0

    anyz = anyzero[0]
    for i in range(1, NROWS):
      anyz = anyz | anyzero[i]
    lax.cond(anyz, trans_all_masked, trans_all_fast)

    for i in R:
      pltpu.make_async_copy(rowhist[i], out_hbm.at[base + i],
                            osem[i]).start()
    return 0

  lax.fori_loop(0, ngroups, pair_body, 0)
  last_base = wid * rows_per_worker + NROWS * (ngroups - 1)
  for i in R:
    pltpu.make_async_copy(rowhist[i], out_hbm.at[last_base + i],
                          osem[i]).wait()


@jax.jit
def kernel(x):
  N, C, Lx = x.shape
  rows = N * C
  x2 = x.reshape(rows, Lx)
  mesh = plsc.VectorSubcoreMesh(core_axis_name="c", subcore_axis_name="s",
                                num_cores=NCORES, num_subcores=NSUB)
  per_row_scratch = [
      pltpu.VMEM((HIST_PAD,), jnp.float32),
      pltpu.VMEM((L,), jnp.int32),
      pltpu.VMEM((L,), jnp.int32),
      pltpu.VMEM((L,), jnp.int32),
      pltpu.VMEM((L,), jnp.int32),
      pltpu.VMEM((RAD0,), jnp.int32),
      pltpu.VMEM((RAD1,), jnp.int32),
      pltpu.VMEM((L + NLANE,), jnp.int32),
  ]
  run = functools.partial(
      pl.kernel,
      mesh=mesh,
      compiler_params=pltpu.CompilerParams(needs_layout_passes=False),
      out_type=jax.ShapeDtypeStruct((rows, HIST_PAD), jnp.float32),
      scratch_types=per_row_scratch * NROWS +
      [pltpu.SemaphoreType.DMA] * (2 * NROWS),
  )(_row_kernel)
  out = run(x2)
  return out[:, :NB * NB].reshape(N, C, NB, NB)


# speedup vs baseline: 1.6261x; 1.0428x over previous
import functools

import jax
import jax.numpy as jnp
from jax import lax
from jax.experimental import pallas as pl
from jax.experimental.pallas import tpu as pltpu
from jax.experimental.pallas import tpu_sc as plsc

L = 4096
NB = 65
HIST_PAD = 4240
NLANE = 16
NVREG = L // NLANE
NCORES = 2
NSUB = 16
NWORKERS = NCORES * NSUB
RAD0 = 1 << 11
RAD1 = 1 << 10
RAD2 = 1 << 11
NROWS = 4

_I32_MIN = -2147483648
_I32_MAX = 2147483647
_NREFS = 9


def _row_kernel(x_hbm, out_hbm, *scratch):
  total_rows = x_hbm.shape[0]
  rows_per_worker = total_rows // NWORKERS
  ngroups = rows_per_worker // NROWS
  wid = lax.axis_index("s") * NCORES + lax.axis_index("c")
  iota = lax.iota(jnp.int32, NLANE)
  zeros16i = jnp.zeros((NLANE,), jnp.int32)
  zeros16f = jnp.zeros((NLANE,), jnp.float32)
  R = range(NROWS)

  xv = [scratch[i * _NREFS + 0] for i in R]
  keys0 = [scratch[i * _NREFS + 1] for i in R]
  pay0 = [scratch[i * _NREFS + 2] for i in R]
  keys1 = [scratch[i * _NREFS + 3] for i in R]
  pay1 = [scratch[i * _NREFS + 4] for i in R]
  hist0 = [scratch[i * _NREFS + 5] for i in R]
  hist1 = [scratch[i * _NREFS + 6] for i in R]
  bins = [scratch[i * _NREFS + 7] for i in R]
  rowhist = [scratch[i * _NREFS + 8] for i in R]
  isem = [scratch[NROWS * _NREFS + i] for i in R]
  osem = [scratch[NROWS * _NREFS + NROWS + i] for i in R]

  for i in R:
    bins[i][pl.ds(L, NLANE)] = zeros16i

  def ds(j):
    return pl.ds(j * NLANE, NLANE)

  def pair_body(r, _):
    base = wid * rows_per_worker + NROWS * r

    for i in R:
      pltpu.make_async_copy(x_hbm.at[base + i], xv[i], isem[i]).wait()

    def h0clear_body(j, _):
      for i in R:
        hist0[i][ds(j)] = zeros16i
      return 0

    lax.fori_loop(0, RAD0 // NLANE, h0clear_body, 0, unroll=4)

    def keyfast_body(j, zs):
      idxv = j * NLANE + iota
      vs = [xv[i][ds(j)] for i in R]
      ts = [plsc.bitcast(v, jnp.int32) for v in vs]
      us = [(t ^ (lax.shift_right_arithmetic(t, 31) & _I32_MAX)) ^ _I32_MIN
            for t in ts]
      dd = [u & (RAD0 - 1) for u in us]
      sc = [plsc.scan_count(d) for d in dd]
      for i in R:
        keys0[i][ds(j)] = us[i]
      for i in R:
        plsc.addupdate_scatter(hist0[i], [dd[i]], sc[i][0], mask=sc[i][1])
      return tuple(zs[i] | (ts[i] + ts[i] == 0) for i in R)

    zs = lax.fori_loop(0, NVREG, keyfast_body, tuple(iota < 0 for _ in R),
                       unroll=2)

    def make_slow_path(i):
      def slow_path():
        def valid_body(j, carry):
          fv, lv = carry
          v = xv[i][ds(j)]
          nz = v != 0.0
          idxv = j * NLANE + iota
          fv = jnp.minimum(fv, jnp.where(nz, idxv, jnp.int32(L)))
          lv = jnp.maximum(lv, jnp.where(nz, idxv, jnp.int32(-1)))
          return fv, lv

        fv, lv = lax.fori_loop(0, NVREG, valid_body,
                               (zeros16i + L, zeros16i - 1), unroll=4)
        s_, e_ = jnp.min(fv), jnp.max(lv)

        def hclear(j, _):
          hist0[i][ds(j)] = zeros16i
          return 0

        lax.fori_loop(0, RAD0 // NLANE, hclear, 0, unroll=4)

        def keymask_body(j, _):
          u = keys0[i][ds(j)]
          idxv = j * NLANE + iota
          ok = (idxv >= s_) & (idxv <= e_)
          key = jnp.where(ok, u, jnp.int32(-1))
          keys0[i][ds(j)] = key
          d = key & (RAD0 - 1)
          occ, last = plsc.scan_count(d)
          plsc.addupdate_scatter(hist0[i], [d], occ, mask=last)
          return 0

        lax.fori_loop(0, NVREG, keymask_body, 0, unroll=4)
        return s_, e_

      return slow_path

    full = lambda: (jnp.int32(0), jnp.int32(L - 1))
    anyzero = [jnp.max(z.astype(jnp.int32)) > 0 for z in zs]
    se = [lax.cond(anyzero[i], make_slow_path(i), full) for i in R]
    start = [s for s, _ in se]
    end = [e for _, e in se]
    vlen = [end[i] - start[i] + 1 for i in R]
    lenc = [jnp.maximum(v, 1) for v in vlen]

    @pl.when(r < ngroups - 1)
    def _():
      for i in R:
        pltpu.make_async_copy(x_hbm.at[base + NROWS + i], xv[i],
                              isem[i]).start()

    def exclusive_scan(srcs, clrs, n, clr_vregs):
      fifteen = zeros16i + (NLANE - 1)
      iters = n // NLANE

      def body(j, carry):
        vs = [src[ds(j)] for src in srcs]
        incs = [plsc.cumsum(v) for v in vs]
        for i in R:
          srcs[i][ds(j)] = incs[i] - vs[i] + carry[i]
        if clr_vregs >= iters:
          k = clr_vregs // iters
          for c in range(k):
            for clr in clrs:
              clr[ds(k * j + c)] = zeros16i
        elif clr_vregs:
          @pl.when(j < clr_vregs)
          def _():
            for clr in clrs:
              clr[ds(j)] = zeros16i
        tots = [jnp.take_along_axis(inc, fifteen, axis=0) for inc in incs]
        return tuple(carry[i] + tots[i] for i in R)

      lax.fori_loop(0, iters, body, tuple(zeros16i for _ in R), unroll=2)

    exclusive_scan(hist0, hist1, RAD0, RAD1 // NLANE)

    def permute01_body(j, sh, nbits, shn, nbitsn, kin, pin, kout, pout,
                       hist, histnext):
      ks = [kin[i][ds(j)] for i in R]
      if pin is None:
        idxv = j * NLANE + iota
        ps = [idxv for _ in R]
      else:
        ps = [pin[i][ds(j)] for i in R]
      dd = [lax.shift_right_logical(k, sh) & ((1 << nbits) - 1) for k in ks]
      dn = [lax.shift_right_logical(k, shn) & ((1 << nbitsn) - 1)
            for k in ks]
      sc = [plsc.scan_count(d) for d in dd]
      scn = [plsc.scan_count(d) for d in dn]
      bases = [plsc.load_gather(hist[i], [dd[i]]) for i in R]
      slots = [bases[i] + sc[i][0] - 1 for i in R]
      for i in R:
        plsc.store_scatter(kout[i], [slots[i]], ks[i])
      for i in R:
        plsc.store_scatter(pout[i], [slots[i]], ps[i])
      for i in R:
        plsc.addupdate_scatter(hist[i], [dd[i]], sc[i][0], mask=sc[i][1])
      for i in R:
        plsc.addupdate_scatter(histnext[i], [dn[i]], scn[i][0],
                               mask=scn[i][1])
      return 0

    lax.fori_loop(
        0, NVREG,
        lambda j, _: permute01_body(j, 0, 11, 11, 10, keys0, None, keys1,
                                    pay1, hist0, hist1),
        0, unroll=2)

    exclusive_scan(hist1, hist0, RAD1, RAD0 // NLANE)

    lax.fori_loop(
        0, NVREG,
        lambda j, _: permute01_body(j, 11, 10, 21, 11, keys1, pay1, keys0,
                                    pay0, hist1, hist0),
        0, unroll=2)

    exclusive_scan(hist0, (), RAD2, 0)

    @pl.when(r > 0)
    def _():
      for i in R:
        pltpu.make_async_copy(rowhist[i], out_hbm.at[base - NROWS + i],
                              osem[i]).wait()

    def hclear_body(j, _):
      for i in R:
        rowhist[i][ds(j)] = zeros16f
      return 0

    lax.fori_loop(0, HIST_PAD // NLANE, hclear_body, 0, unroll=4)

    invlen = [(zeros16f + 1.0) / (zeros16i + lenc[i]).astype(jnp.float32)
              for i in R]

    def permute2_body(j, _):
      ks = [keys0[i][ds(j)] for i in R]
      ps = [pay0[i][ds(j)] for i in R]
      dd = [lax.shift_right_logical(k, 21) & (RAD2 - 1) for k in ks]
      sc = [plsc.scan_count(d) for d in dd]
      bases = [plsc.load_gather(hist0[i], [dd[i]]) for i in R]
      slots = [bases[i] + sc[i][0] - 1 for i in R]
      for i in R:
        plsc.addupdate_scatter(hist0[i], [dd[i]], sc[i][0], mask=sc[i][1])
      bf = [(slots[i] * NB).astype(jnp.float32) * invlen[i] + 5e-5
            for i in R]
      bb = [jnp.minimum(b.astype(jnp.int32), NB - 1) for b in bf]
      for i in R:
        plsc.store_scatter(bins[i], [ps[i]], bb[i])
      return 0

    lax.fori_loop(0, NVREG, permute2_body, 0, unroll=2)

    inv = [(zeros16f + 1.0) /
           (zeros16i + jnp.maximum(vlen[i] - 1, 1)).astype(jnp.float32)
           for i in R]

    def trans_masked_one(j, i):
      a = bins[i][ds(j)]
      b = bins[i][pl.ds(j * NLANE + 1, NLANE)]
      t = j * NLANE + iota
      ok = (t >= start[i]) & (t <= end[i] - 1)
      cell = a * NB + b
      occ, last = plsc.scan_count(cell, mask=ok)
      plsc.addupdate_scatter(rowhist[i], [cell],
                             occ.astype(jnp.float32) * inv[i],
                             mask=last & ok)
      return 0

    def trans_all_masked():
      def body(j, _):
        for i in R:
          trans_masked_one(j, i)
        return 0

      lax.fori_loop(0, NVREG, body, 0, unroll=2)
      return 0

    def trans_all_fast():
      def body(j, _):
        aa = [bins[i][ds(j)] for i in R]
        ab = [bins[i][pl.ds(j * NLANE + 1, NLANE)] for i in R]
        cells = [aa[i] * NB + ab[i] for i in R]
        sc = [plsc.scan_count(c) for c in cells]
        for i in R:
          plsc.addupdate_scatter(rowhist[i], [cells[i]],
                                 sc[i][0].astype(jnp.float32) * inv[i],
                                 mask=sc[i][1])
        return 0

      lax.fori_loop(0, NVREG - 1, body, 0, unroll=2)
      for i in R:
        trans_masked_one(NVREG - 1, i)
      return 0

    anyz = anyzero[0]
    for i in range(1, NROWS):
      anyz = anyz | anyzero[i]
    lax.cond(anyz, trans_all_masked, trans_all_fast)

    for i in R:
      pltpu.make_async_copy(rowhist[i], out_hbm.at[base + i],
                            osem[i]).start()
    return 0

  first_base = wid * rows_per_worker
  for i in R:
    pltpu.make_async_copy(x_hbm.at[first_base + i], xv[i], isem[i]).start()
  lax.fori_loop(0, ngroups, pair_body, 0)
  last_base = wid * rows_per_worker + NROWS * (ngroups - 1)
  for i in R:
    pltpu.make_async_copy(rowhist[i], out_hbm.at[last_base + i],
                          osem[i]).wait()


@jax.jit
def kernel(x):
  N, C, Lx = x.shape
  rows = N * C
  x2 = x.reshape(rows, Lx)
  mesh = plsc.VectorSubcoreMesh(core_axis_name="c", subcore_axis_name="s",
                                num_cores=NCORES, num_subcores=NSUB)
  per_row_scratch = [
      pltpu.VMEM((L,), jnp.float32),
      pltpu.VMEM((L,), jnp.int32),
      pltpu.VMEM((L,), jnp.int32),
      pltpu.VMEM((L,), jnp.int32),
      pltpu.VMEM((L,), jnp.int32),
      pltpu.VMEM((RAD0,), jnp.int32),
      pltpu.VMEM((RAD1,), jnp.int32),
      pltpu.VMEM((L + NLANE,), jnp.int32),
      pltpu.VMEM((HIST_PAD,), jnp.float32),
  ]
  run = functools.partial(
      pl.kernel,
      mesh=mesh,
      compiler_params=pltpu.CompilerParams(needs_layout_passes=False),
      out_type=jax.ShapeDtypeStruct((rows, HIST_PAD), jnp.float32),
      scratch_types=per_row_scratch * NROWS +
      [pltpu.SemaphoreType.DMA] * (2 * NROWS),
  )(_row_kernel)
  out = run(x2)
  return out[:, :NB * NB].reshape(N, C, NB, NB)


# speedup vs baseline: 1.6405x; 1.0089x over previous
import functools

import jax
import jax.numpy as jnp
from jax import lax
from jax.experimental import pallas as pl
from jax.experimental.pallas import tpu as pltpu
from jax.experimental.pallas import tpu_sc as plsc

L = 4096
NB = 65
HIST_PAD = 4240
NLANE = 16
NVREG = L // NLANE
NCORES = 2
NSUB = 16
NWORKERS = NCORES * NSUB
RAD0 = 1 << 11
RAD1 = 1 << 10
RAD2 = 1 << 11
NROWS = 4

_I32_MIN = -2147483648
_I32_MAX = 2147483647
_NREFS = 9


def _row_kernel(x_hbm, out_hbm, *scratch):
  total_rows = x_hbm.shape[0]
  rows_per_worker = total_rows // NWORKERS
  ngroups = rows_per_worker // NROWS
  wid = lax.axis_index("s") * NCORES + lax.axis_index("c")
  iota = lax.iota(jnp.int32, NLANE)
  zeros16i = jnp.zeros((NLANE,), jnp.int32)
  zeros16f = jnp.zeros((NLANE,), jnp.float32)
  R = range(NROWS)

  xv = [scratch[i * _NREFS + 0] for i in R]
  keys0 = [scratch[i * _NREFS + 1] for i in R]
  pay0 = [scratch[i * _NREFS + 2] for i in R]
  keys1 = [scratch[i * _NREFS + 3] for i in R]
  pay1 = [scratch[i * _NREFS + 4] for i in R]
  hist0 = [scratch[i * _NREFS + 5] for i in R]
  hist1 = [scratch[i * _NREFS + 6] for i in R]
  bins = [scratch[i * _NREFS + 7] for i in R]
  rowhist = [scratch[i * _NREFS + 8] for i in R]
  isem = [scratch[NROWS * _NREFS + i] for i in R]
  osem = [scratch[NROWS * _NREFS + NROWS + i] for i in R]

  for i in R:
    bins[i][pl.ds(L, NLANE)] = zeros16i

  def ds(j):
    return pl.ds(j * NLANE, NLANE)

  def pair_body(r, _):
    base = wid * rows_per_worker + NROWS * r

    for i in R:
      pltpu.make_async_copy(x_hbm.at[base + i], xv[i], isem[i]).wait()

    def h0clear_body(j, _):
      for i in R:
        hist0[i][ds(j)] = zeros16i
      return 0

    lax.fori_loop(0, RAD0 // NLANE, h0clear_body, 0, unroll=4)

    def keyfast_body(j, zs):
      idxv = j * NLANE + iota
      vs = [xv[i][ds(j)] for i in R]
      ts = [plsc.bitcast(v, jnp.int32) for v in vs]
      us = [t ^ (lax.shift_right_arithmetic(t, 31) | _I32_MIN) for t in ts]
      dd = [u & (RAD0 - 1) for u in us]
      sc = [plsc.scan_count(d) for d in dd]
      for i in R:
        keys0[i][ds(j)] = us[i]
      for i in R:
        plsc.addupdate_scatter(hist0[i], [dd[i]], sc[i][0], mask=sc[i][1])
      return tuple(zs[i] | (ts[i] + ts[i] == 0) for i in R)

    zs = lax.fori_loop(0, NVREG, keyfast_body, tuple(iota < 0 for _ in R),
                       unroll=2)

    def make_slow_path(i):
      def slow_path():
        def valid_body(j, carry):
          fv, lv = carry
          v = xv[i][ds(j)]
          nz = v != 0.0
          idxv = j * NLANE + iota
          fv = jnp.minimum(fv, jnp.where(nz, idxv, jnp.int32(L)))
          lv = jnp.maximum(lv, jnp.where(nz, idxv, jnp.int32(-1)))
          return fv, lv

        fv, lv = lax.fori_loop(0, NVREG, valid_body,
                               (zeros16i + L, zeros16i - 1), unroll=4)
        s_, e_ = jnp.min(fv), jnp.max(lv)

        def hclear(j, _):
          hist0[i][ds(j)] = zeros16i
          return 0

        lax.fori_loop(0, RAD0 // NLANE, hclear, 0, unroll=4)

        def keymask_body(j, _):
          u = keys0[i][ds(j)]
          idxv = j * NLANE + iota
          ok = (idxv >= s_) & (idxv <= e_)
          key = jnp.where(ok, u, jnp.int32(-1))
          keys0[i][ds(j)] = key
          d = key & (RAD0 - 1)
          occ, last = plsc.scan_count(d)
          plsc.addupdate_scatter(hist0[i], [d], occ, mask=last)
          return 0

        lax.fori_loop(0, NVREG, keymask_body, 0, unroll=4)
        return s_, e_

      return slow_path

    full = lambda: (jnp.int32(0), jnp.int32(L - 1))
    anyzero = [jnp.max(z.astype(jnp.int32)) > 0 for z in zs]
    se = [lax.cond(anyzero[i], make_slow_path(i), full) for i in R]
    start = [s for s, _ in se]
    end = [e for _, e in se]
    vlen = [end[i] - start[i] + 1 for i in R]
    lenc = [jnp.maximum(v, 1) for v in vlen]

    @pl.when(r < ngroups - 1)
    def _():
      for i in R:
        pltpu.make_async_copy(x_hbm.at[base + NROWS + i], xv[i],
                              isem[i]).start()

    def exclusive_scan(srcs, clrs, n, clr_vregs):
      fifteen = zeros16i + (NLANE - 1)
      iters = n // NLANE

      def body(j, carry):
        vs = [src[ds(j)] for src in srcs]
        incs = [plsc.cumsum(v) for v in vs]
        for i in R:
          srcs[i][ds(j)] = incs[i] - vs[i] + carry[i]
        if clr_vregs >= iters:
          k = clr_vregs // iters
          for c in range(k):
            for clr in clrs:
              clr[ds(k * j + c)] = zeros16i
        elif clr_vregs:
          @pl.when(j < clr_vregs)
          def _():
            for clr in clrs:
              clr[ds(j)] = zeros16i
        tots = [jnp.take_along_axis(inc, fifteen, axis=0) for inc in incs]
        return tuple(carry[i] + tots[i] for i in R)

      lax.fori_loop(0, iters, body, tuple(zeros16i for _ in R), unroll=2)

    exclusive_scan(hist0, hist1, RAD0, RAD1 // NLANE)

    def permute01_body(j, sh, nbits, shn, nbitsn, kin, pin, kout, pout,
                       hist, histnext):
      ks = [kin[i][ds(j)] for i in R]
      if pin is None:
        idxv = j * NLANE + iota
        ps = [idxv for _ in R]
      else:
        ps = [pin[i][ds(j)] for i in R]
      dd = [lax.shift_right_logical(k, sh) & ((1 << nbits) - 1) for k in ks]
      dn = [lax.shift_right_logical(k, shn) & ((1 << nbitsn) - 1)
            for k in ks]
      sc = [plsc.scan_count(d) for d in dd]
      scn = [plsc.scan_count(d) for d in dn]
      bases = [plsc.load_gather(hist[i], [dd[i]]) for i in R]
      slots = [bases[i] + sc[i][0] - 1 for i in R]
      for i in R:
        plsc.store_scatter(kout[i], [slots[i]], ks[i])
      for i in R:
        plsc.store_scatter(pout[i], [slots[i]], ps[i])
      for i in R:
        plsc.addupdate_scatter(hist[i], [dd[i]], sc[i][0], mask=sc[i][1])
      for i in R:
        plsc.addupdate_scatter(histnext[i], [dn[i]], scn[i][0],
                               mask=scn[i][1])
      return 0

    lax.fori_loop(
        0, NVREG,
        lambda j, _: permute01_body(j, 0, 11, 11, 10, keys0, None, keys1,
                                    pay1, hist0, hist1),
        0, unroll=2)

    exclusive_scan(hist1, hist0, RAD1, RAD0 // NLANE)

    lax.fori_loop(
        0, NVREG,
        lambda j, _: permute01_body(j, 11, 10, 21, 11, keys1, pay1, keys0,
                                    pay0, hist1, hist0),
        0, unroll=2)

    exclusive_scan(hist0, (), RAD2, 0)

    @pl.when(r > 0)
    def _():
      for i in R:
        pltpu.make_async_copy(rowhist[i], out_hbm.at[base - NROWS + i],
                              osem[i]).wait()

    def hclear_body(j, _):
      for i in R:
        rowhist[i][ds(j)] = zeros16f
      return 0

    lax.fori_loop(0, HIST_PAD // NLANE, hclear_body, 0, unroll=4)

    invlen = [(zeros16f + float(NB)) /
              (zeros16i + lenc[i]).astype(jnp.float32) for i in R]

    def permute2_body(j, _):
      ks = [keys0[i][ds(j)] for i in R]
      ps = [pay0[i][ds(j)] for i in R]
      dd = [lax.shift_right_logical(k, 21) & (RAD2 - 1) for k in ks]
      sc = [plsc.scan_count(d) for d in dd]
      bases = [plsc.load_gather(hist0[i], [dd[i]]) for i in R]
      slots = [bases[i] + sc[i][0] - 1 for i in R]
      for i in R:
        plsc.addupdate_scatter(hist0[i], [dd[i]], sc[i][0], mask=sc[i][1])
      bf = [slots[i].astype(jnp.float32) * invlen[i] + 5e-5 for i in R]
      bb = [jnp.minimum(b.astype(jnp.int32), NB - 1) for b in bf]
      for i in R:
        plsc.store_scatter(bins[i], [ps[i]], bb[i])
      return 0

    lax.fori_loop(0, NVREG, permute2_body, 0, unroll=2)

    inv = [(zeros16f + 1.0) /
           (zeros16i + jnp.maximum(vlen[i] - 1, 1)).astype(jnp.float32)
           for i in R]

    def trans_masked_one(j, i):
      a = bins[i][ds(j)]
      b = bins[i][pl.ds(j * NLANE + 1, NLANE)]
      t = j * NLANE + iota
      ok = (t >= start[i]) & (t <= end[i] - 1)
      cell = a * NB + b
      occ, last = plsc.scan_count(cell, mask=ok)
      plsc.addupdate_scatter(rowhist[i], [cell],
                             occ.astype(jnp.float32) * inv[i],
                             mask=last & ok)
      return 0

    def trans_all_masked():
      def body(j, _):
        for i in R:
          trans_masked_one(j, i)
        return 0

      lax.fori_loop(0, NVREG, body, 0, unroll=2)
      return 0

    def trans_all_fast():
      def body(j, _):
        aa = [bins[i][ds(j)] for i in R]
        ab = [bins[i][pl.ds(j * NLANE + 1, NLANE)] for i in R]
        cells = [aa[i] * NB + ab[i] for i in R]
        sc = [plsc.scan_count(c) for c in cells]
        for i in R:
          plsc.addupdate_scatter(rowhist[i], [cells[i]],
                                 sc[i][0].astype(jnp.float32) * inv[i],
                                 mask=sc[i][1])
        return 0

      lax.fori_loop(0, NVREG - 1, body, 0, unroll=2)
      for i in R:
        trans_masked_one(NVREG - 1, i)
      return 0

    anyz = anyzero[0]
    for i in range(1, NROWS):
      anyz = anyz | anyzero[i]
    lax.cond(anyz, trans_all_masked, trans_all_fast)

    for i in R:
      pltpu.make_async_copy(rowhist[i], out_hbm.at[base + i],
                            osem[i]).start()
    return 0

  first_base = wid * rows_per_worker
  for i in R:
    pltpu.make_async_copy(x_hbm.at[first_base + i], xv[i], isem[i]).start()
  lax.fori_loop(0, ngroups, pair_body, 0)
  last_base = wid * rows_per_worker + NROWS * (ngroups - 1)
  for i in R:
    pltpu.make_async_copy(rowhist[i], out_hbm.at[last_base + i],
                          osem[i]).wait()


@jax.jit
def kernel(x):
  N, C, Lx = x.shape
  rows = N * C
  x2 = x.reshape(rows, Lx)
  mesh = plsc.VectorSubcoreMesh(core_axis_name="c", subcore_axis_name="s",
                                num_cores=NCORES, num_subcores=NSUB)
  per_row_scratch = [
      pltpu.VMEM((L,), jnp.float32),
      pltpu.VMEM((L,), jnp.int32),
      pltpu.VMEM((L,), jnp.int32),
      pltpu.VMEM((L,), jnp.int32),
      pltpu.VMEM((L,), jnp.int32),
      pltpu.VMEM((RAD0,), jnp.int32),
      pltpu.VMEM((RAD1,), jnp.int32),
      pltpu.VMEM((L + NLANE,), jnp.int32),
      pltpu.VMEM((HIST_PAD,), jnp.float32),
  ]
  run = functools.partial(
      pl.kernel,
      mesh=mesh,
      compiler_params=pltpu.CompilerParams(needs_layout_passes=False),
      out_type=jax.ShapeDtypeStruct((rows, HIST_PAD), jnp.float32),
      scratch_types=per_row_scratch * NROWS +
      [pltpu.SemaphoreType.DMA] * (2 * NROWS),
  )(_row_kernel)
  out = run(x2)
  return out[:, :NB * NB].reshape(N, C, NB, NB)
